# baseline (speedup 1.0000x reference)
import jax
import jax.numpy as jnp
from jax.experimental import pallas as pl
from jax.experimental.pallas import tpu as pltpu

_BI = 1024
_NJ = 2048


def _bf16_split2(v):
    hi = v.astype(jnp.bfloat16).astype(jnp.float32)
    mid = (v - hi).astype(jnp.bfloat16).astype(jnp.float32)
    return hi, mid


def _chamfer_block_kernel(gts_ref, predsT_ref, out_ref,
                          yaug_ref, colmin_ref, rowacc_ref):
    i = pl.program_id(1)
    n_i = pl.num_programs(1)
    N = predsT_ref.shape[2]
    BI = gts_ref.shape[1]

    @pl.when(i == 0)
    def _():
        yT = predsT_ref[0]
        ry = jnp.sum(yT * yT, axis=0, keepdims=True)
        ry_hi, ry_mid = _bf16_split2(ry)
        yaug_ref[...] = jnp.concatenate(
            [yT, jnp.ones((2, N), jnp.float32), ry_hi, ry_mid],
            axis=0).astype(jnp.bfloat16)
        rowacc_ref[0] = 0.0
        colmin_ref[...] = jnp.full_like(colmin_ref[...], jnp.inf)

    x = gts_ref[0]
    rx = jnp.sum(x * x, axis=1, keepdims=True)
    rx_hi, rx_mid = _bf16_split2(rx)
    xaug = jnp.concatenate(
        [x * -2.0, rx_hi, rx_mid,
         jnp.ones((x.shape[0], 2), jnp.float32)],
        axis=1).astype(jnp.bfloat16)

    row_part = jnp.full((BI, _NJ), jnp.inf, jnp.float32)
    for j in range(0, N, _NJ):
        p = jax.lax.dot_general(
            xaug, yaug_ref[:, j:j + _NJ], (((1,), (0,)), ((), ())),
            preferred_element_type=jnp.float32)
        row_part = jnp.minimum(row_part, p)
        colmin_ref[0:1, j:j + _NJ] = jnp.minimum(
            colmin_ref[0:1, j:j + _NJ], jnp.min(p, axis=0, keepdims=True))

    rowacc_ref[0] += jnp.sum(jnp.min(row_part, axis=1))

    @pl.when(i == n_i - 1)
    def _():
        total = rowacc_ref[0] + jnp.sum(colmin_ref[...])
        out_ref[...] = jnp.full((1, 1, 1), total, dtype=jnp.float32)


def _chamfer(preds, gts, interpret=False):
    B, N, D = preds.shape
    predsT = jnp.transpose(preds, (0, 2, 1))
    out = pl.pallas_call(
        _chamfer_block_kernel,
        out_shape=jax.ShapeDtypeStruct((B, 1, 1), jnp.float32),
        grid=(B, N // _BI),
        in_specs=[
            pl.BlockSpec((1, _BI, D), lambda b, i: (b, i, 0)),
            pl.BlockSpec((1, D, N), lambda b, i: (b, 0, 0)),
        ],
        out_specs=pl.BlockSpec((1, 1, 1), lambda b, i: (b, 0, 0)),
        scratch_shapes=[
            pltpu.VMEM((D + 4, N), jnp.bfloat16),
            pltpu.VMEM((1, N), jnp.float32),
            pltpu.SMEM((1,), jnp.float32),
        ],
        compiler_params=pltpu.CompilerParams(
            dimension_semantics=("parallel", "arbitrary"),
            vmem_limit_bytes=56 * 1024 * 1024,
        ),
        name="chamfer_loss",
        interpret=interpret,
    )(gts, predsT)
    return jnp.sum(out)


def kernel(preds, gts):
    return _chamfer(preds, gts)


# speedup vs baseline: 1.0452x; 1.0452x over previous
import jax
import jax.numpy as jnp
from jax.experimental import pallas as pl
from jax.experimental.pallas import tpu as pltpu

_BI = 2048
_NJ = 2048


def _bf16_split2(v):
    hi = v.astype(jnp.bfloat16).astype(jnp.float32)
    mid = (v - hi).astype(jnp.bfloat16).astype(jnp.float32)
    return hi, mid


def _chamfer_block_kernel(gts_ref, predsT_ref, out_ref,
                          yaug_ref, colmin_ref, rowacc_ref):
    i = pl.program_id(1)
    n_i = pl.num_programs(1)
    N = predsT_ref.shape[2]
    BI = gts_ref.shape[1]

    @pl.when(i == 0)
    def _():
        yT = predsT_ref[0]
        ry = jnp.sum(yT * yT, axis=0, keepdims=True)
        ry_hi, ry_mid = _bf16_split2(ry)
        yaug_ref[...] = jnp.concatenate(
            [yT, jnp.ones((2, N), jnp.float32), ry_hi, ry_mid],
            axis=0).astype(jnp.bfloat16)
        rowacc_ref[0] = 0.0
        colmin_ref[...] = jnp.full_like(colmin_ref[...], jnp.inf)

    x = gts_ref[0]
    rx = jnp.sum(x * x, axis=1, keepdims=True)
    rx_hi, rx_mid = _bf16_split2(rx)
    xaug = jnp.concatenate(
        [x * -2.0, rx_hi, rx_mid,
         jnp.ones((x.shape[0], 2), jnp.float32)],
        axis=1).astype(jnp.bfloat16)

    row_part = jnp.full((BI, _NJ), jnp.inf, jnp.float32)
    for j in range(0, N, _NJ):
        p = jax.lax.dot_general(
            xaug, yaug_ref[:, j:j + _NJ], (((1,), (0,)), ((), ())),
            preferred_element_type=jnp.float32)
        row_part = jnp.minimum(row_part, p)
        colmin_ref[0:1, j:j + _NJ] = jnp.minimum(
            colmin_ref[0:1, j:j + _NJ], jnp.min(p, axis=0, keepdims=True))

    rowacc_ref[0] += jnp.sum(jnp.min(row_part, axis=1))

    @pl.when(i == n_i - 1)
    def _():
        total = rowacc_ref[0] + jnp.sum(colmin_ref[...])
        out_ref[...] = jnp.full((1, 1, 1), total, dtype=jnp.float32)


def _chamfer(preds, gts, interpret=False):
    B, N, D = preds.shape
    predsT = jnp.transpose(preds, (0, 2, 1))
    out = pl.pallas_call(
        _chamfer_block_kernel,
        out_shape=jax.ShapeDtypeStruct((B, 1, 1), jnp.float32),
        grid=(B, N // _BI),
        in_specs=[
            pl.BlockSpec((1, _BI, D), lambda b, i: (b, i, 0)),
            pl.BlockSpec((1, D, N), lambda b, i: (b, 0, 0)),
        ],
        out_specs=pl.BlockSpec((1, 1, 1), lambda b, i: (b, 0, 0)),
        scratch_shapes=[
            pltpu.VMEM((D + 4, N), jnp.bfloat16),
            pltpu.VMEM((1, N), jnp.float32),
            pltpu.SMEM((1,), jnp.float32),
        ],
        compiler_params=pltpu.CompilerParams(
            dimension_semantics=("parallel", "arbitrary"),
            vmem_limit_bytes=56 * 1024 * 1024,
        ),
        name="chamfer_loss",
        interpret=interpret,
    )(gts, predsT)
    return jnp.sum(out)


def kernel(preds, gts):
    return _chamfer(preds, gts)
